# baseline (speedup 1.0000x reference)
import jax
import jax.numpy as jnp
from jax import lax
from jax.experimental import pallas as pl
from jax.experimental.pallas import tpu as pltpu
from jax.experimental.pallas import tpu_sc as plsc

NC = 2
NS = 16
L = 16
B = 16384
D = 64
V = 100000
GF = 8
PHASES = D // (NC * GF)
LCHUNK = 1024
NCHUNK = B // LCHUNK
MI_PER_TILE = GF * NCHUNK // NS


def _bpr_body(ut_hbm, it_hbm, uidx_hbm, pidx_hbm, nidx_hbm, out_hbm,
              slabrow, idxa, ubuf, dbuf, acc,
              u_pair, d_pair, sem):
    c = lax.axis_index("c")
    s = lax.axis_index("s")

    zeros = jnp.zeros((L,), jnp.float32)

    def zero_body(vo, cc):
        for k in range(8):
            acc[pl.ds((vo * 8 + k) * L, L)] = zeros
        return cc

    lax.fori_loop(0, MI_PER_TILE * LCHUNK // L // 8, zero_body, 0)

    def phase_body(p, carry):
        d = c * (D // NC) + p * GF + lax.rem(s, GF)

        @pl.when(s < GF)
        def _():
            pltpu.sync_copy(ut_hbm.at[d, pl.ds(0, V)], slabrow)

            def chunk_u(ch, cc):
                pltpu.sync_copy(uidx_hbm.at[pl.ds(ch * LCHUNK, LCHUNK)], idxa)

                def vb(vo, cc2):
                    for k in range(8):
                        o = (vo * 8 + k) * L
                        ubuf[pl.ds(o, L)] = plsc.load_gather(
                            slabrow, [idxa[pl.ds(o, L)]])
                    return cc2

                lax.fori_loop(0, LCHUNK // L // 8, vb, 0)
                pltpu.sync_copy(
                    ubuf, u_pair.at[pl.ds(s * B + ch * LCHUNK, LCHUNK)])
                return cc

            lax.fori_loop(0, NCHUNK, chunk_u, 0)

        @pl.when(s >= GF)
        def _():
            si = s - GF
            pltpu.sync_copy(it_hbm.at[d, pl.ds(0, V)], slabrow)

            def chunk_i(ch, cc):
                pltpu.sync_copy(pidx_hbm.at[pl.ds(ch * LCHUNK, LCHUNK)], idxa)

                def vbp(vo, cc2):
                    for k in range(8):
                        o = (vo * 8 + k) * L
                        ubuf[pl.ds(o, L)] = plsc.load_gather(
                            slabrow, [idxa[pl.ds(o, L)]])
                    return cc2

                lax.fori_loop(0, LCHUNK // L // 8, vbp, 0)
                pltpu.sync_copy(nidx_hbm.at[pl.ds(ch * LCHUNK, LCHUNK)], idxa)

                def vbn(vo, cc2):
                    for k in range(8):
                        o = (vo * 8 + k) * L
                        ubuf[pl.ds(o, L)] = (
                            ubuf[pl.ds(o, L)]
                            - plsc.load_gather(slabrow, [idxa[pl.ds(o, L)]]))
                    return cc2

                lax.fori_loop(0, LCHUNK // L // 8, vbn, 0)
                pltpu.sync_copy(
                    ubuf, d_pair.at[pl.ds(si * B + ch * LCHUNK, LCHUNK)])
                return cc

            lax.fori_loop(0, NCHUNK, chunk_i, 0)

        plsc.subcore_barrier()

        f = s // 2
        for li in range(MI_PER_TILE):
            ch = (s % 2) * MI_PER_TILE + li
            off = pl.multiple_of(f * B + ch * LCHUNK, 8)
            cu = pltpu.async_copy(u_pair.at[pl.ds(off, LCHUNK)], ubuf, sem)
            cd = pltpu.async_copy(d_pair.at[pl.ds(off, LCHUNK)], dbuf, sem)
            cu.wait()
            cd.wait()

            def mul_body(vo, cc2):
                for k in range(8):
                    o = (vo * 8 + k) * L
                    lo = li * LCHUNK + o
                    acc[pl.ds(lo, L)] = (acc[pl.ds(lo, L)]
                                         + ubuf[pl.ds(o, L)]
                                         * dbuf[pl.ds(o, L)])
                return cc2

            lax.fori_loop(0, LCHUNK // L // 8, mul_body, 0)
        plsc.subcore_barrier()
        return carry

    lax.fori_loop(0, PHASES, phase_body, 0)

    f = s // 2
    for li in range(MI_PER_TILE):
        ch = (s % 2) * MI_PER_TILE + li
        off = pl.multiple_of(f * B + ch * LCHUNK, 8)
        pltpu.sync_copy(acc.at[pl.ds(li * LCHUNK, LCHUNK)],
                        u_pair.at[pl.ds(off, LCHUNK)])
    plsc.subcore_barrier()

    b0 = pl.multiple_of(s * 1024, 8)
    pltpu.sync_copy(u_pair.at[pl.ds(b0, 1024)], ubuf.at[pl.ds(0, 1024)])
    for ff in range(1, GF):
        pltpu.sync_copy(u_pair.at[pl.ds(ff * B + b0, 1024)],
                        dbuf.at[pl.ds(0, 1024)])

        def add_body(vo, cc2, _ff=ff):
            for k in range(8):
                o = (vo * 8 + k) * L
                ubuf[pl.ds(o, L)] = ubuf[pl.ds(o, L)] + dbuf[pl.ds(o, L)]
            return cc2

        lax.fori_loop(0, 1024 // L // 8, add_body, 0)
    pltpu.sync_copy(
        ubuf.at[pl.ds(0, 1024)],
        out_hbm.at[pl.ds(pl.multiple_of(c * B + s * 1024, 8), 1024)])


@jax.jit
def _bpr_sc(ut_t, it_t, uidx, pidx, nidx):
    mesh = plsc.VectorSubcoreMesh(
        core_axis_name="c", subcore_axis_name="s", num_cores=NC, num_subcores=NS
    )
    return pl.kernel(
        _bpr_body,
        out_type=jax.ShapeDtypeStruct((NC * B,), jnp.float32),
        mesh=mesh,
        scratch_types=[
            pltpu.VMEM((V,), jnp.float32),
            pltpu.VMEM((LCHUNK,), jnp.int32),
            pltpu.VMEM((LCHUNK,), jnp.float32),
            pltpu.VMEM((LCHUNK,), jnp.float32),
            pltpu.VMEM((MI_PER_TILE * LCHUNK,), jnp.float32),
            pltpu.VMEM_SHARED((GF * B,), jnp.float32),
            pltpu.VMEM_SHARED((GF * B,), jnp.float32),
            pltpu.SemaphoreType.DMA,
        ],
        compiler_params=pltpu.CompilerParams(
            needs_layout_passes=False, use_tc_tiling_on_sc=False),
    )(ut_t, it_t, uidx, pidx, nidx)


def _fin_body(p_ref, o_ref):
    o_ref[...] = p_ref[pl.ds(0, B)] + p_ref[pl.ds(B, B)]


@jax.jit
def _finish(partials):
    return pl.pallas_call(
        _fin_body,
        out_shape=jax.ShapeDtypeStruct((B,), jnp.float32),
    )(partials)


def kernel(user_table, item_table, user_input, pos_item_input, neg_item_input):
    partials = _bpr_sc(user_table.T, item_table.T,
                       user_input.reshape(-1).astype(jnp.int32),
                       pos_item_input.reshape(-1).astype(jnp.int32),
                       neg_item_input.reshape(-1).astype(jnp.int32))
    return _finish(partials).reshape(B, 1)


# speedup vs baseline: 4.1802x; 4.1802x over previous
import jax
import jax.numpy as jnp
from jax import lax
from jax.experimental import pallas as pl
from jax.experimental.pallas import tpu as pltpu
from jax.experimental.pallas import tpu_sc as plsc

NC = 2
NS = 16
L = 16
NW = NC * NS

B = 16384
D = 64
CHUNK = 128
B_PER_W = B // NW
NCHUNK = B_PER_W // CHUNK
GROUPS_PER_CHUNK = CHUNK // L


def _bpr_body(u_tab, i_tab, uidx_hbm, pidx_hbm, nidx_hbm, out_hbm,
              uidx_v, pidx_v, nidx_v, urows, prows, nrows, out_v,
              isem, gsem):
    wid = lax.axis_index("s") * NC + lax.axis_index("c")
    base = pl.multiple_of(wid * B_PER_W, 8)

    idx_copies = [
        pltpu.async_copy(uidx_hbm.at[pl.ds(base, B_PER_W)], uidx_v, isem),
        pltpu.async_copy(pidx_hbm.at[pl.ds(base, B_PER_W)], pidx_v, isem),
        pltpu.async_copy(nidx_hbm.at[pl.ds(base, B_PER_W)], nidx_v, isem),
    ]
    for c in idx_copies:
        c.wait()

    copies = []
    for j in range(NCHUNK):
        src = pl.ds(j * CHUNK, CHUNK)
        dst = pl.ds(j * CHUNK, CHUNK)
        copies.append(pltpu.async_copy(
            u_tab.at[uidx_v.at[src]], urows.at[dst], gsem))
        copies.append(pltpu.async_copy(
            i_tab.at[pidx_v.at[src]], prows.at[dst], gsem))
        copies.append(pltpu.async_copy(
            i_tab.at[nidx_v.at[src]], nrows.at[dst], gsem))

    iota = lax.iota(jnp.int32, L)

    def group_body(g, carry):
        rb = g * L
        rowids = rb + iota
        acc0 = jnp.zeros((L,), jnp.float32)
        acc1 = jnp.zeros((L,), jnp.float32)
        for d in range(0, D, 2):
            dv0 = jnp.full((L,), d, jnp.int32)
            dv1 = jnp.full((L,), d + 1, jnp.int32)
            u0 = plsc.load_gather(urows, [rowids, dv0])
            p0 = plsc.load_gather(prows, [rowids, dv0])
            n0 = plsc.load_gather(nrows, [rowids, dv0])
            u1 = plsc.load_gather(urows, [rowids, dv1])
            p1 = plsc.load_gather(prows, [rowids, dv1])
            n1 = plsc.load_gather(nrows, [rowids, dv1])
            acc0 = acc0 + u0 * (p0 - n0)
            acc1 = acc1 + u1 * (p1 - n1)
        out_v[pl.ds(rb, L)] = acc0 + acc1
        return carry

    for j in range(NCHUNK):
        for c in copies[3 * j:3 * j + 3]:
            c.wait()
        lax.fori_loop(j * GROUPS_PER_CHUNK, (j + 1) * GROUPS_PER_CHUNK,
                      group_body, 0)

    pltpu.sync_copy(out_v, out_hbm.at[pl.ds(base, B_PER_W)])


@jax.jit
def _bpr_sc(user_table, item_table, uidx, pidx, nidx):
    mesh = plsc.VectorSubcoreMesh(
        core_axis_name="c", subcore_axis_name="s", num_cores=NC, num_subcores=NS
    )
    return pl.kernel(
        _bpr_body,
        out_type=jax.ShapeDtypeStruct((B,), jnp.float32),
        mesh=mesh,
        scratch_types=[
            pltpu.VMEM((B_PER_W,), jnp.int32),
            pltpu.VMEM((B_PER_W,), jnp.int32),
            pltpu.VMEM((B_PER_W,), jnp.int32),
            pltpu.VMEM((B_PER_W, D), jnp.float32),
            pltpu.VMEM((B_PER_W, D), jnp.float32),
            pltpu.VMEM((B_PER_W, D), jnp.float32),
            pltpu.VMEM((B_PER_W,), jnp.float32),
            pltpu.SemaphoreType.DMA,
            pltpu.SemaphoreType.DMA,
        ],
        compiler_params=pltpu.CompilerParams(
            needs_layout_passes=False, use_tc_tiling_on_sc=False),
    )(user_table, item_table, uidx, pidx, nidx)


def kernel(user_table, item_table, user_input, pos_item_input, neg_item_input):
    out = _bpr_sc(user_table, item_table,
                  user_input.reshape(-1).astype(jnp.int32),
                  pos_item_input.reshape(-1).astype(jnp.int32),
                  neg_item_input.reshape(-1).astype(jnp.int32))
    return out.reshape(B, 1)
